# baseline (speedup 1.0000x reference)
import functools

import jax
import jax.numpy as jnp
from jax import lax
from jax.experimental import pallas as pl
from jax.experimental.pallas import tpu as pltpu
from jax.experimental.pallas import tpu_sc as plsc
from jax._src.pallas import mpmd as _mpmd

D = 32
LANES = 16
NW = 32
B = 16384
F = 26
BF = B * F
ROWS128 = BF // 128
PER_W_ROWS = ROWS128 // NW
SUB = 4
CHUNK = SUB * 128
N_CHUNKS = PER_W_ROWS // SUB
NBUF = 3

_DTYPE = jnp.bfloat16

_MESH = plsc.VectorSubcoreMesh(core_axis_name="c", subcore_axis_name="s")
_SCRATCH = [
    pltpu.VMEM((PER_W_ROWS, 128), jnp.int32),
    pltpu.VMEM((PER_W_ROWS, 128), jnp.int32),
    pltpu.VMEM((CHUNK, D), _DTYPE),
    pltpu.VMEM((CHUNK, D), _DTYPE),
    pltpu.VMEM((CHUNK, D), _DTYPE),
    pltpu.SemaphoreType.DMA,
    pltpu.SemaphoreType.DMA,
]


def _half_body(parity):
    def k(tab_hbm, idx_hbm, *rest):
        if parity == 0:
            out_hbm, idx_all, dst_all, b0, b1, b2, gsem, ssem = rest
        else:
            _, out_hbm, idx_all, dst_all, b0, b1, b2, gsem, ssem = rest
        bufs = (b0, b1, b2)
        wid = lax.axis_index("s") * 2 + lax.axis_index("c")
        row0 = wid * PER_W_ROWS
        pltpu.sync_copy(idx_hbm.at[pl.ds(row0, PER_W_ROWS)], idx_all)

        lane2 = lax.broadcasted_iota(jnp.int32, (LANES,), 0) * 2

        def build_row(r, carry):
            base = (row0 + r) * 256 + parity
            for t in range(128 // LANES):
                dst_all[r, pl.ds(t * LANES, LANES)] = lane2 + (base + 2 * t * LANES)
            return carry

        lax.fori_loop(0, PER_W_ROWS, build_row, 0)

        def gather_copies(g, buf):
            for j in range(SUB):
                yield pltpu.make_async_copy(
                    tab_hbm.at[idx_all.at[g * SUB + j]],
                    buf.at[pl.ds(j * 128, 128)], gsem)

        def scatter_copies(g, buf):
            for j in range(SUB):
                yield pltpu.make_async_copy(
                    buf.at[pl.ds(j * 128, 128)],
                    out_hbm.at[dst_all.at[g * SUB + j]], ssem)

        for g in range(N_CHUNKS):
            b = g % NBUF
            if g >= NBUF:
                for c in scatter_copies(g - NBUF, bufs[b]):
                    c.wait()
            for c in gather_copies(g, bufs[b]):
                c.start()
            if g >= 1:
                pb = (g - 1) % NBUF
                for c in gather_copies(g - 1, bufs[pb]):
                    c.wait()
                for c in scatter_copies(g - 1, bufs[pb]):
                    c.start()
        lb_ = (N_CHUNKS - 1) % NBUF
        for c in gather_copies(N_CHUNKS - 1, bufs[lb_]):
            c.wait()
        for c in scatter_copies(N_CHUNKS - 1, bufs[lb_]):
            c.start()
        for g in range(N_CHUNKS - NBUF, N_CHUNKS):
            for c in scatter_copies(g, bufs[g % NBUF]):
                c.wait()

    return k


_OUT_T = jax.ShapeDtypeStruct((2 * BF, D), _DTYPE)
_PARAMS = pltpu.CompilerParams(use_tc_tiling_on_sc=False)


def kernel(indices, static_features, learnable_table):
    idx2d = indices.astype(jnp.int32).reshape(ROWS128, 128)
    call_a = _mpmd._mpmd_map(
        [(_MESH, _half_body(0))],
        out_types=_OUT_T,
        scratch_types=_SCRATCH,
        compiler_params=_PARAMS,
        name="hybrid_embed_static",
    )
    out_a = call_a(static_features.astype(_DTYPE), idx2d)
    call_b = _mpmd._mpmd_map(
        [(_MESH, _half_body(1))],
        out_types=_OUT_T,
        scratch_types=_SCRATCH,
        compiler_params=_PARAMS,
        input_output_aliases={2: 0},
        name="hybrid_embed_learn",
    )
    out2 = call_b(learnable_table.astype(_DTYPE), idx2d, out_a)
    return out2.astype(jnp.float32).reshape(B, F, 2 * D)


# speedup vs baseline: 1.5302x; 1.5302x over previous
import functools

import jax
import jax.numpy as jnp
from jax import lax
from jax.experimental import pallas as pl
from jax.experimental.pallas import tpu as pltpu
from jax.experimental.pallas import tpu_sc as plsc
from jax._src.pallas import mpmd as _mpmd

D = 32
LANES = 16
NW = 32
B = 16384
F = 26
BF = B * F
ROWS128 = BF // 128
PER_W_ROWS = ROWS128 // NW
SUB = 4
CHUNK = SUB * 128
N_CHUNKS = PER_W_ROWS // SUB
NBUF = 3

_MESH = plsc.VectorSubcoreMesh(core_axis_name="c", subcore_axis_name="s")
_SCRATCH = [
    pltpu.VMEM((PER_W_ROWS, 128), jnp.int32),
    pltpu.VMEM((PER_W_ROWS, 128), jnp.int32),
    pltpu.VMEM((CHUNK, D), jnp.float32),
    pltpu.VMEM((CHUNK, D), jnp.float32),
    pltpu.VMEM((CHUNK, D), jnp.float32),
    pltpu.SemaphoreType.DMA,
    pltpu.SemaphoreType.DMA,
]


def _half_body(parity):
    def k(tab_hbm, idx_hbm, *rest):
        if parity == 0:
            out_hbm, idx_all, dst_all, b0, b1, b2, gsem, ssem = rest
        else:
            _, out_hbm, idx_all, dst_all, b0, b1, b2, gsem, ssem = rest
        bufs = (b0, b1, b2)
        wid = lax.axis_index("s") * 2 + lax.axis_index("c")
        row0 = wid * PER_W_ROWS
        pltpu.sync_copy(idx_hbm.at[pl.ds(row0, PER_W_ROWS)], idx_all)

        lane2 = lax.broadcasted_iota(jnp.int32, (LANES,), 0) * 2

        def build_row(r, carry):
            base = (row0 + r) * 256 + parity
            for t in range(128 // LANES):
                dst_all[r, pl.ds(t * LANES, LANES)] = lane2 + (base + 2 * t * LANES)
            return carry

        lax.fori_loop(0, PER_W_ROWS, build_row, 0)

        def gather_copies(g, buf):
            for j in range(SUB):
                yield pltpu.make_async_copy(
                    tab_hbm.at[idx_all.at[g * SUB + j]],
                    buf.at[pl.ds(j * 128, 128)], gsem)

        def scatter_copies(g, buf):
            for j in range(SUB):
                yield pltpu.make_async_copy(
                    buf.at[pl.ds(j * 128, 128)],
                    out_hbm.at[dst_all.at[g * SUB + j]], ssem)

        for g in range(N_CHUNKS):
            b = g % NBUF
            if g >= NBUF:
                for c in scatter_copies(g - NBUF, bufs[b]):
                    c.wait()
            for c in gather_copies(g, bufs[b]):
                c.start()
            if g >= 1:
                pb = (g - 1) % NBUF
                for c in gather_copies(g - 1, bufs[pb]):
                    c.wait()
                for c in scatter_copies(g - 1, bufs[pb]):
                    c.start()
        lb_ = (N_CHUNKS - 1) % NBUF
        for c in gather_copies(N_CHUNKS - 1, bufs[lb_]):
            c.wait()
        for c in scatter_copies(N_CHUNKS - 1, bufs[lb_]):
            c.start()
        for g in range(N_CHUNKS - NBUF, N_CHUNKS):
            for c in scatter_copies(g, bufs[g % NBUF]):
                c.wait()

    return k


_OUT_T = jax.ShapeDtypeStruct((2 * BF, D), jnp.float32)
_PARAMS = pltpu.CompilerParams(use_tc_tiling_on_sc=False)


def kernel(indices, static_features, learnable_table):
    idx2d = indices.astype(jnp.int32).reshape(ROWS128, 128)
    call_a = _mpmd._mpmd_map(
        [(_MESH, _half_body(0))],
        out_types=_OUT_T,
        scratch_types=_SCRATCH,
        compiler_params=_PARAMS,
        name="hybrid_embed_static",
    )
    out_a = call_a(static_features, idx2d)
    call_b = _mpmd._mpmd_map(
        [(_MESH, _half_body(1))],
        out_types=_OUT_T,
        scratch_types=_SCRATCH,
        compiler_params=_PARAMS,
        input_output_aliases={2: 0},
        name="hybrid_embed_learn",
    )
    out2 = call_b(learnable_table, idx2d, out_a)
    return out2.reshape(B, F, 2 * D)
